# baseline (speedup 1.0000x reference)
import functools

import jax
import jax.numpy as jnp
from jax import lax
from jax.experimental import pallas as pl
from jax.experimental.pallas import tpu as pltpu
from jax.experimental.pallas import tpu_sc as plsc

N = 10000
E = 320000
D = 128
R = 7
NU = 200
NG = 64

BN = 2000
NB = N // BN

NC = 2
NS = 16
NW = NC * NS
K = 128
NCH0 = 120
NCH1 = 48
TOTCH = NS * (NCH0 + NCH1)
E_PAD = TOTCH * K
CB1 = NS * NCH0
NPS = 624
NTAIL = N - NS * NPS


def _embed_body(ut_ref, emb_ref, wlin_ref, x_ref, y_ref):
    ut = ut_ref[0, 0, :]
    onehot = (ut[:, None] == lax.broadcasted_iota(jnp.int32, (BN, NU), 1))
    x = jnp.dot(onehot.astype(jnp.float32), emb_ref[...],
                preferred_element_type=jnp.float32)
    x_ref[...] = x
    for r in range(R):
        y_ref[r] = jnp.dot(x, wlin_ref[r], preferred_element_type=jnp.float32)


def _embed_call(unit_type, embedding, wlin):
    return pl.pallas_call(
        _embed_body,
        grid=(NB,),
        in_specs=[
            pl.BlockSpec((1, 1, BN), lambda i: (i, 0, 0)),
            pl.BlockSpec((NU, D), lambda i: (0, 0)),
            pl.BlockSpec((R, D, D), lambda i: (0, 0, 0)),
        ],
        out_specs=[
            pl.BlockSpec((BN, D), lambda i: (i, 0)),
            pl.BlockSpec((R, BN, D), lambda i: (0, i, 0)),
        ],
        out_shape=[
            jax.ShapeDtypeStruct((N, D), jnp.float32),
            jax.ShapeDtypeStruct((R, N, D), jnp.float32),
        ],
    )(unit_type.reshape(NB, 1, BN), embedding, wlin)


def _combine_body(agg_ref, x_ref, wself_ref, b_ref, wlin_ref, h_ref, y_ref):
    upd = agg_ref[0] + agg_ref[1]
    h = upd + jnp.dot(x_ref[...], wself_ref[...],
                      preferred_element_type=jnp.float32)
    h = jnp.maximum(h + b_ref[0], 0.0) + x_ref[...]
    h_ref[...] = h
    for r in range(R):
        y_ref[r] = jnp.dot(h, wlin_ref[r], preferred_element_type=jnp.float32)


def _combine_call(agg, x, wself, b, wlin_next):
    return pl.pallas_call(
        _combine_body,
        grid=(NB,),
        in_specs=[
            pl.BlockSpec((2, BN, D), lambda i: (0, i, 0)),
            pl.BlockSpec((BN, D), lambda i: (i, 0)),
            pl.BlockSpec((D, D), lambda i: (0, 0)),
            pl.BlockSpec((1, D), lambda i: (0, 0)),
            pl.BlockSpec((R, D, D), lambda i: (0, 0, 0)),
        ],
        out_specs=[
            pl.BlockSpec((BN, D), lambda i: (i, 0)),
            pl.BlockSpec((R, BN, D), lambda i: (0, i, 0)),
        ],
        out_shape=[
            jax.ShapeDtypeStruct((N, D), jnp.float32),
            jax.ShapeDtypeStruct((R, N, D), jnp.float32),
        ],
    )(agg, x, wself, b, wlin_next)


def _final_body(agg_ref, x_ref, wself_ref, b_ref, n2g_ref, h_ref, gf_ref):
    i = pl.program_id(0)
    upd = agg_ref[0] + agg_ref[1]
    h = upd + jnp.dot(x_ref[...], wself_ref[...],
                      preferred_element_type=jnp.float32)
    h = jnp.maximum(h + b_ref[0], 0.0) + x_ref[...]
    h_ref[...] = h
    n2g = n2g_ref[0, 0, :]
    onehot = (n2g[:, None] == lax.broadcasted_iota(jnp.int32, (BN, NG), 1))
    contrib = lax.dot_general(onehot.astype(jnp.float32), h,
                              (((0,), (0,)), ((), ())),
                              preferred_element_type=jnp.float32)

    @pl.when(i == 0)
    def _():
        gf_ref[...] = jnp.zeros_like(gf_ref)

    gf_ref[...] += contrib


def _final_call(agg, x, wself, b, node2graph):
    return pl.pallas_call(
        _final_body,
        grid=(NB,),
        in_specs=[
            pl.BlockSpec((2, BN, D), lambda i: (0, i, 0)),
            pl.BlockSpec((BN, D), lambda i: (i, 0)),
            pl.BlockSpec((D, D), lambda i: (0, 0)),
            pl.BlockSpec((1, D), lambda i: (0, 0)),
            pl.BlockSpec((1, 1, BN), lambda i: (i, 0, 0)),
        ],
        out_specs=[
            pl.BlockSpec((BN, D), lambda i: (i, 0)),
            pl.BlockSpec((NG, D), lambda i: (0, 0)),
        ],
        out_shape=[
            jax.ShapeDtypeStruct((N, D), jnp.float32),
            jax.ShapeDtypeStruct((NG, D), jnp.float32),
        ],
    )(agg, x, wself, b, node2graph.reshape(NB, 1, BN))


_sc_mesh = plsc.VectorSubcoreMesh(core_axis_name="c", subcore_axis_name="s")


_DNUMS = lax.GatherDimensionNumbers(
    offset_dims=(), collapsed_slice_dims=(0,), start_index_map=(0,))


@functools.partial(
    pl.kernel,
    mesh=_sc_mesh,
    out_type=jax.ShapeDtypeStruct((NC * N, D), jnp.float32),
    scratch_types=[
        pltpu.VMEM((4, 2, K), jnp.int32),
        pltpu.VMEM((4, K), jnp.float32),
        pltpu.VMEM((3, K, D), jnp.float32),
        pltpu.VMEM_SHARED((N, D), jnp.float32),
    ] + [pltpu.SemaphoreType.DMA] * 7,
)
def _sc_aggregate(edata_hbm, ew_hbm, y_hbm, zrow_hbm, out_hbm,
                  edata_v, ew_v, rows_v, acc_sh,
                  rsem0, rsem1, rsem2, isem0, isem1, isem2, isem3):
    cid = lax.axis_index("c")
    sid = lax.axis_index("s")
    rsems = (rsem0, rsem1, rsem2)
    isems = (isem0, isem1, isem2, isem3)
    nch = jnp.where(cid == 0, NCH0, NCH1)
    cbase = jnp.where(cid == 0, sid * NCH0, CB1 + sid * NCH1)

    pltpu.sync_copy(zrow_hbm, acc_sh.at[pl.ds(sid * NPS, NPS)])

    @pl.when(sid == NS - 1)
    def _():
        pltpu.sync_copy(zrow_hbm.at[pl.ds(0, NTAIL)],
                        acc_sh.at[pl.ds(NS * NPS, NTAIL)])

    plsc.subcore_barrier()

    def fetch(c, ib):
        pltpu.async_copy(edata_hbm.at[cbase + c], edata_v.at[ib], isems[ib])
        pltpu.async_copy(ew_hbm.at[cbase + c], ew_v.at[ib], isems[ib])

    def wait_fetch(ib):
        pltpu.make_async_copy(edata_hbm.at[0], edata_v.at[ib],
                              isems[ib]).wait()
        pltpu.make_async_copy(ew_hbm.at[0], ew_v.at[ib],
                              isems[ib]).wait()

    def issue(ib, rb):
        pltpu.async_copy(y_hbm.at[edata_v.at[ib, 0]], rows_v.at[rb],
                         rsems[rb])

    def wait_rows(rb):
        pltpu.make_async_copy(y_hbm.at[pl.ds(0, K)], rows_v.at[rb],
                              rsems[rb]).wait()

    def scale(ib, rb):
        def scale_body(g, c2):
            ewv = ew_v[ib, pl.ds(g * 16, 16)]
            for l in range(16):
                w = lax.gather(ewv, jnp.full((16, 1), l, jnp.int32), _DNUMS,
                               slice_sizes=(1,),
                               mode=lax.GatherScatterMode.PROMISE_IN_BOUNDS)
                for j in range(D // 16):
                    rows_v[rb, g * 16 + l, pl.ds(j * 16, 16)] = \
                        rows_v[rb, g * 16 + l, pl.ds(j * 16, 16)] * w
            return c2

        lax.fori_loop(0, K // 16, scale_body, 0)

    def scatter(ib, rb):
        pltpu.sync_copy(rows_v.at[rb], acc_sh.at[edata_v.at[ib, 1]],
                        add=True)

    for ib in range(4):
        fetch(jnp.int32(ib), ib)
    for ib in range(3):
        wait_fetch(ib)
        issue(ib, ib)

    def twelve_body(t, carry):
        c0 = 12 * t
        for u in range(12):
            c = c0 + u
            ib = u % 4
            rb = u % 3
            wait_rows(rb)
            scale(ib, rb)
            scatter(ib, rb)

            @pl.when(c + 3 < nch)
            def _():
                wait_fetch((u + 3) % 4)
                issue((u + 3) % 4, rb)

            @pl.when(c + 4 < nch)
            def _():
                fetch(c + 4, ib)

        return carry

    lax.fori_loop(0, nch // 12, twelve_body, 0)
    plsc.subcore_barrier()
    pltpu.sync_copy(acc_sh.at[pl.ds(sid * NPS, NPS)],
                    out_hbm.at[pl.ds(cid * N + sid * NPS, NPS)])

    @pl.when(sid == NS - 1)
    def _():
        pltpu.sync_copy(acc_sh.at[pl.ds(NS * NPS, NTAIL)],
                        out_hbm.at[pl.ds(cid * N + NS * NPS, NTAIL)])


def _aggregate(gidx, dst, ew, y):
    pad = E_PAD - E
    gidx_p = jnp.concatenate([gidx, jnp.zeros((pad,), jnp.int32)])
    dst_p = jnp.concatenate([dst.astype(jnp.int32), jnp.zeros((pad,), jnp.int32)])
    ew_p = jnp.concatenate([ew, jnp.zeros((pad,), jnp.float32)])
    edata = jnp.stack([gidx_p.reshape(TOTCH, K),
                       dst_p.reshape(TOTCH, K)], axis=1)
    zrow = jnp.zeros((NPS, D), jnp.float32)
    out = _sc_aggregate(edata, ew_p.reshape(TOTCH, K),
                        y.reshape(R * N, D), zrow)
    return out.reshape(2, N, D)


def kernel(unit_type, edge_index, edge_relation, edge_weight, node2graph,
           embedding,
           W_lin0, b_lin0, W_self0, b_self0,
           W_lin1, b_lin1, W_self1, b_self1,
           W_lin2, b_lin2, W_self2, b_self2):
    src = edge_index[0]
    dst = edge_index[1]
    gidx = edge_relation.astype(jnp.int32) * N + src.astype(jnp.int32)

    wl0 = W_lin0.reshape(R, D, D)
    wl1 = W_lin1.reshape(R, D, D)
    wl2 = W_lin2.reshape(R, D, D)
    b0 = (b_lin0 + b_self0).reshape(1, D)
    b1 = (b_lin1 + b_self1).reshape(1, D)
    b2 = (b_lin2 + b_self2).reshape(1, D)

    x0, y0 = _embed_call(unit_type.astype(jnp.int32), embedding, wl0)
    agg0 = _aggregate(gidx, dst, edge_weight, y0)
    x1, y1 = _combine_call(agg0, x0, W_self0, b0, wl1)
    agg1 = _aggregate(gidx, dst, edge_weight, y1)
    x2, y2 = _combine_call(agg1, x1, W_self1, b1, wl2)
    agg2 = _aggregate(gidx, dst, edge_weight, y2)
    node_feature, graph_feature = _final_call(agg2, x2, W_self2, b2,
                                              node2graph.astype(jnp.int32))
    return graph_feature, node_feature


# speedup vs baseline: 2.3138x; 2.3138x over previous
import functools

import jax
import jax.numpy as jnp
from jax import lax
from jax.experimental import pallas as pl
from jax.experimental.pallas import tpu as pltpu
from jax.experimental.pallas import tpu_sc as plsc

N = 10000
E = 320000
D = 128
R = 7
NU = 200
NG = 64

BN = 2000
NB = N // BN

NC = 2
NS = 16
NW = NC * NS
K = 128
NCH0 = 108
NCH1 = 52
TOTCH = NS * (NCH0 + NCH1)
E_PAD = TOTCH * K
CB1 = NS * NCH0
NPS = 624
NTAIL = N - NS * NPS


def _embed_body(ut_ref, emb_ref, wlin_ref, x_ref, y_ref):
    ut = ut_ref[0, 0, :]
    onehot = (ut[:, None] == lax.broadcasted_iota(jnp.int32, (BN, NU), 1))
    x = jnp.dot(onehot.astype(jnp.float32), emb_ref[...],
                preferred_element_type=jnp.float32)
    x_ref[...] = x
    for r in range(R):
        y_ref[r] = jnp.dot(x, wlin_ref[r], preferred_element_type=jnp.float32)


def _embed_call(unit_type, embedding, wlin):
    return pl.pallas_call(
        _embed_body,
        grid=(NB,),
        in_specs=[
            pl.BlockSpec((1, 1, BN), lambda i: (i, 0, 0)),
            pl.BlockSpec((NU, D), lambda i: (0, 0)),
            pl.BlockSpec((R, D, D), lambda i: (0, 0, 0)),
        ],
        out_specs=[
            pl.BlockSpec((BN, D), lambda i: (i, 0)),
            pl.BlockSpec((R, BN, D), lambda i: (0, i, 0)),
        ],
        out_shape=[
            jax.ShapeDtypeStruct((N, D), jnp.float32),
            jax.ShapeDtypeStruct((R, N, D), jnp.float32),
        ],
    )(unit_type.reshape(NB, 1, BN), embedding, wlin)


def _combine_body(agg_ref, x_ref, wself_ref, b_ref, wlin_ref, h_ref, y_ref):
    upd = agg_ref[0] + agg_ref[1]
    h = upd + jnp.dot(x_ref[...], wself_ref[...],
                      preferred_element_type=jnp.float32)
    h = jnp.maximum(h + b_ref[0], 0.0) + x_ref[...]
    h_ref[...] = h
    for r in range(R):
        y_ref[r] = jnp.dot(h, wlin_ref[r], preferred_element_type=jnp.float32)


def _combine_call(agg, x, wself, b, wlin_next):
    return pl.pallas_call(
        _combine_body,
        grid=(NB,),
        in_specs=[
            pl.BlockSpec((2, BN, D), lambda i: (0, i, 0)),
            pl.BlockSpec((BN, D), lambda i: (i, 0)),
            pl.BlockSpec((D, D), lambda i: (0, 0)),
            pl.BlockSpec((1, D), lambda i: (0, 0)),
            pl.BlockSpec((R, D, D), lambda i: (0, 0, 0)),
        ],
        out_specs=[
            pl.BlockSpec((BN, D), lambda i: (i, 0)),
            pl.BlockSpec((R, BN, D), lambda i: (0, i, 0)),
        ],
        out_shape=[
            jax.ShapeDtypeStruct((N, D), jnp.float32),
            jax.ShapeDtypeStruct((R, N, D), jnp.float32),
        ],
    )(agg, x, wself, b, wlin_next)


def _final_body(agg_ref, x_ref, wself_ref, b_ref, n2g_ref, h_ref, gf_ref):
    i = pl.program_id(0)
    upd = agg_ref[0] + agg_ref[1]
    h = upd + jnp.dot(x_ref[...], wself_ref[...],
                      preferred_element_type=jnp.float32)
    h = jnp.maximum(h + b_ref[0], 0.0) + x_ref[...]
    h_ref[...] = h
    n2g = n2g_ref[0, 0, :]
    onehot = (n2g[:, None] == lax.broadcasted_iota(jnp.int32, (BN, NG), 1))
    contrib = lax.dot_general(onehot.astype(jnp.float32), h,
                              (((0,), (0,)), ((), ())),
                              preferred_element_type=jnp.float32)

    @pl.when(i == 0)
    def _():
        gf_ref[...] = jnp.zeros_like(gf_ref)

    gf_ref[...] += contrib


def _final_call(agg, x, wself, b, node2graph):
    return pl.pallas_call(
        _final_body,
        grid=(NB,),
        in_specs=[
            pl.BlockSpec((2, BN, D), lambda i: (0, i, 0)),
            pl.BlockSpec((BN, D), lambda i: (i, 0)),
            pl.BlockSpec((D, D), lambda i: (0, 0)),
            pl.BlockSpec((1, D), lambda i: (0, 0)),
            pl.BlockSpec((1, 1, BN), lambda i: (i, 0, 0)),
        ],
        out_specs=[
            pl.BlockSpec((BN, D), lambda i: (i, 0)),
            pl.BlockSpec((NG, D), lambda i: (0, 0)),
        ],
        out_shape=[
            jax.ShapeDtypeStruct((N, D), jnp.float32),
            jax.ShapeDtypeStruct((NG, D), jnp.float32),
        ],
    )(agg, x, wself, b, node2graph.reshape(NB, 1, BN))


_sc_mesh = plsc.VectorSubcoreMesh(core_axis_name="c", subcore_axis_name="s")


_DNUMS = lax.GatherDimensionNumbers(
    offset_dims=(), collapsed_slice_dims=(0,), start_index_map=(0,))


@functools.partial(
    pl.kernel,
    mesh=_sc_mesh,
    out_type=jax.ShapeDtypeStruct((NC * N, D), jnp.float32),
    scratch_types=[
        pltpu.VMEM((4, 2, K), jnp.int32),
        pltpu.VMEM((4, K), jnp.float32),
        pltpu.VMEM((2, K, D), jnp.float32),
        pltpu.VMEM_SHARED((N, D), jnp.float32),
    ] + [pltpu.SemaphoreType.DMA] * 6,
)
def _sc_aggregate(edata_hbm, ew_hbm, y_hbm, zrow_hbm, out_hbm,
                  edata_v, ew_v, rows_v, acc_sh,
                  rsem0, rsem1, isem0, isem1, isem2, isem3):
    cid = lax.axis_index("c")
    sid = lax.axis_index("s")
    rsems = (rsem0, rsem1)
    isems = (isem0, isem1, isem2, isem3)
    nch = jnp.where(cid == 0, NCH0, NCH1)
    cbase = jnp.where(cid == 0, sid * NCH0, CB1 + sid * NCH1)

    pltpu.sync_copy(zrow_hbm, acc_sh.at[pl.ds(sid * NPS, NPS)])

    @pl.when(sid == NS - 1)
    def _():
        pltpu.sync_copy(zrow_hbm.at[pl.ds(0, NTAIL)],
                        acc_sh.at[pl.ds(NS * NPS, NTAIL)])

    plsc.subcore_barrier()

    def fetch(c, ib):
        pltpu.async_copy(edata_hbm.at[cbase + c], edata_v.at[ib], isems[ib])
        pltpu.async_copy(ew_hbm.at[cbase + c], ew_v.at[ib], isems[ib])

    def wait_fetch(ib):
        pltpu.make_async_copy(edata_hbm.at[0], edata_v.at[ib],
                              isems[ib]).wait()
        pltpu.make_async_copy(ew_hbm.at[0], ew_v.at[ib],
                              isems[ib]).wait()

    def issue(ib, rb):
        pltpu.async_copy(y_hbm.at[edata_v.at[ib, 0]], rows_v.at[rb],
                         rsems[rb])

    def wait_rows(rb):
        pltpu.make_async_copy(y_hbm.at[pl.ds(0, K)], rows_v.at[rb],
                              rsems[rb]).wait()

    def scale(ib, rb):
        def scale_body(g, c2):
            ewv = ew_v[ib, pl.ds(g * 16, 16)]
            for l in range(16):
                w = lax.gather(ewv, jnp.full((16, 1), l, jnp.int32), _DNUMS,
                               slice_sizes=(1,),
                               mode=lax.GatherScatterMode.PROMISE_IN_BOUNDS)
                for j in range(D // 16):
                    rows_v[rb, g * 16 + l, pl.ds(j * 16, 16)] = \
                        rows_v[rb, g * 16 + l, pl.ds(j * 16, 16)] * w
            return c2

        lax.fori_loop(0, K // 16, scale_body, 0)

    def scatter(ib, rb):
        pltpu.sync_copy(rows_v.at[rb], acc_sh.at[edata_v.at[ib, 1]],
                        add=True)

    for ib in range(4):
        fetch(jnp.int32(ib), ib)
    for ib in range(2):
        wait_fetch(ib)
        issue(ib, ib)

    def quad_body(t, carry):
        c0 = 4 * t
        for u in range(4):
            c = c0 + u
            ib = u
            rb = u % 2
            wait_rows(rb)
            scale(ib, rb)
            scatter(ib, rb)

            @pl.when(c + 2 < nch)
            def _():
                wait_fetch((u + 2) % 4)
                issue((u + 2) % 4, rb)

            @pl.when(c + 4 < nch)
            def _():
                fetch(c + 4, ib)

        return carry

    lax.fori_loop(0, nch // 4, quad_body, 0)
    plsc.subcore_barrier()
    pltpu.sync_copy(acc_sh.at[pl.ds(sid * NPS, NPS)],
                    out_hbm.at[pl.ds(cid * N + sid * NPS, NPS)])

    @pl.when(sid == NS - 1)
    def _():
        pltpu.sync_copy(acc_sh.at[pl.ds(NS * NPS, NTAIL)],
                        out_hbm.at[pl.ds(cid * N + NS * NPS, NTAIL)])


def _aggregate(gidx, dst, ew, y):
    pad = E_PAD - E
    gidx_p = jnp.concatenate([gidx, jnp.zeros((pad,), jnp.int32)])
    dst_p = jnp.concatenate([dst.astype(jnp.int32), jnp.zeros((pad,), jnp.int32)])
    ew_p = jnp.concatenate([ew, jnp.zeros((pad,), jnp.float32)])
    edata = jnp.stack([gidx_p.reshape(TOTCH, K),
                       dst_p.reshape(TOTCH, K)], axis=1)
    zrow = jnp.zeros((NPS, D), jnp.float32)
    out = _sc_aggregate(edata, ew_p.reshape(TOTCH, K),
                        y.reshape(R * N, D), zrow)
    return out.reshape(2, N, D)


def kernel(unit_type, edge_index, edge_relation, edge_weight, node2graph,
           embedding,
           W_lin0, b_lin0, W_self0, b_self0,
           W_lin1, b_lin1, W_self1, b_self1,
           W_lin2, b_lin2, W_self2, b_self2):
    src = edge_index[0]
    dst = edge_index[1]
    gidx = edge_relation.astype(jnp.int32) * N + src.astype(jnp.int32)

    wl0 = W_lin0.reshape(R, D, D)
    wl1 = W_lin1.reshape(R, D, D)
    wl2 = W_lin2.reshape(R, D, D)
    b0 = (b_lin0 + b_self0).reshape(1, D)
    b1 = (b_lin1 + b_self1).reshape(1, D)
    b2 = (b_lin2 + b_self2).reshape(1, D)

    x0, y0 = _embed_call(unit_type.astype(jnp.int32), embedding, wl0)
    agg0 = _aggregate(gidx, dst, edge_weight, y0)
    x1, y1 = _combine_call(agg0, x0, W_self0, b0, wl1)
    agg1 = _aggregate(gidx, dst, edge_weight, y1)
    x2, y2 = _combine_call(agg1, x1, W_self1, b1, wl2)
    agg2 = _aggregate(gidx, dst, edge_weight, y2)
    node_feature, graph_feature = _final_call(agg2, x2, W_self2, b2,
                                              node2graph.astype(jnp.int32))
    return graph_feature, node_feature


# speedup vs baseline: 2.3276x; 1.0060x over previous
import functools

import jax
import jax.numpy as jnp
from jax import lax
from jax.experimental import pallas as pl
from jax.experimental.pallas import tpu as pltpu
from jax.experimental.pallas import tpu_sc as plsc

N = 10000
E = 320000
D = 128
R = 7
NU = 200
NG = 64

BN = 2000
NB = N // BN

NC = 2
NS = 16
NW = NC * NS
K = 128
NCH0 = 116
NCH1 = 44
TOTCH = NS * (NCH0 + NCH1)
E_PAD = TOTCH * K
CB1 = NS * NCH0
NPS = 624
NTAIL = N - NS * NPS


def _embed_body(ut_ref, emb_ref, wlin_ref, x_ref, y_ref):
    ut = ut_ref[0, 0, :]
    onehot = (ut[:, None] == lax.broadcasted_iota(jnp.int32, (BN, NU), 1))
    x = jnp.dot(onehot.astype(jnp.float32), emb_ref[...],
                preferred_element_type=jnp.float32)
    x_ref[...] = x
    for r in range(R):
        y_ref[r] = jnp.dot(x, wlin_ref[r], preferred_element_type=jnp.float32)


def _embed_call(unit_type, embedding, wlin):
    return pl.pallas_call(
        _embed_body,
        grid=(NB,),
        in_specs=[
            pl.BlockSpec((1, 1, BN), lambda i: (i, 0, 0)),
            pl.BlockSpec((NU, D), lambda i: (0, 0)),
            pl.BlockSpec((R, D, D), lambda i: (0, 0, 0)),
        ],
        out_specs=[
            pl.BlockSpec((BN, D), lambda i: (i, 0)),
            pl.BlockSpec((R, BN, D), lambda i: (0, i, 0)),
        ],
        out_shape=[
            jax.ShapeDtypeStruct((N, D), jnp.float32),
            jax.ShapeDtypeStruct((R, N, D), jnp.float32),
        ],
    )(unit_type.reshape(NB, 1, BN), embedding, wlin)


def _combine_body(agg_ref, x_ref, wself_ref, b_ref, wlin_ref, h_ref, y_ref):
    upd = agg_ref[0] + agg_ref[1]
    h = upd + jnp.dot(x_ref[...], wself_ref[...],
                      preferred_element_type=jnp.float32)
    h = jnp.maximum(h + b_ref[0], 0.0) + x_ref[...]
    h_ref[...] = h
    for r in range(R):
        y_ref[r] = jnp.dot(h, wlin_ref[r], preferred_element_type=jnp.float32)


def _combine_call(agg, x, wself, b, wlin_next):
    return pl.pallas_call(
        _combine_body,
        grid=(NB,),
        in_specs=[
            pl.BlockSpec((2, BN, D), lambda i: (0, i, 0)),
            pl.BlockSpec((BN, D), lambda i: (i, 0)),
            pl.BlockSpec((D, D), lambda i: (0, 0)),
            pl.BlockSpec((1, D), lambda i: (0, 0)),
            pl.BlockSpec((R, D, D), lambda i: (0, 0, 0)),
        ],
        out_specs=[
            pl.BlockSpec((BN, D), lambda i: (i, 0)),
            pl.BlockSpec((R, BN, D), lambda i: (0, i, 0)),
        ],
        out_shape=[
            jax.ShapeDtypeStruct((N, D), jnp.float32),
            jax.ShapeDtypeStruct((R, N, D), jnp.float32),
        ],
    )(agg, x, wself, b, wlin_next)


def _final_body(agg_ref, x_ref, wself_ref, b_ref, n2g_ref, h_ref, gf_ref):
    i = pl.program_id(0)
    upd = agg_ref[0] + agg_ref[1]
    h = upd + jnp.dot(x_ref[...], wself_ref[...],
                      preferred_element_type=jnp.float32)
    h = jnp.maximum(h + b_ref[0], 0.0) + x_ref[...]
    h_ref[...] = h
    n2g = n2g_ref[0, 0, :]
    onehot = (n2g[:, None] == lax.broadcasted_iota(jnp.int32, (BN, NG), 1))
    contrib = lax.dot_general(onehot.astype(jnp.float32), h,
                              (((0,), (0,)), ((), ())),
                              preferred_element_type=jnp.float32)

    @pl.when(i == 0)
    def _():
        gf_ref[...] = jnp.zeros_like(gf_ref)

    gf_ref[...] += contrib


def _final_call(agg, x, wself, b, node2graph):
    return pl.pallas_call(
        _final_body,
        grid=(NB,),
        in_specs=[
            pl.BlockSpec((2, BN, D), lambda i: (0, i, 0)),
            pl.BlockSpec((BN, D), lambda i: (i, 0)),
            pl.BlockSpec((D, D), lambda i: (0, 0)),
            pl.BlockSpec((1, D), lambda i: (0, 0)),
            pl.BlockSpec((1, 1, BN), lambda i: (i, 0, 0)),
        ],
        out_specs=[
            pl.BlockSpec((BN, D), lambda i: (i, 0)),
            pl.BlockSpec((NG, D), lambda i: (0, 0)),
        ],
        out_shape=[
            jax.ShapeDtypeStruct((N, D), jnp.float32),
            jax.ShapeDtypeStruct((NG, D), jnp.float32),
        ],
    )(agg, x, wself, b, node2graph.reshape(NB, 1, BN))


_sc_mesh = plsc.VectorSubcoreMesh(core_axis_name="c", subcore_axis_name="s")


_DNUMS = lax.GatherDimensionNumbers(
    offset_dims=(), collapsed_slice_dims=(0,), start_index_map=(0,))


@functools.partial(
    pl.kernel,
    mesh=_sc_mesh,
    out_type=jax.ShapeDtypeStruct((NC * N, D), jnp.float32),
    scratch_types=[
        pltpu.VMEM((4, 2, K), jnp.int32),
        pltpu.VMEM((4, K), jnp.float32),
        pltpu.VMEM((2, K, D), jnp.float32),
        pltpu.VMEM_SHARED((N, D), jnp.float32),
    ] + [pltpu.SemaphoreType.DMA] * 6,
)
def _sc_aggregate(edata_hbm, ew_hbm, y_hbm, zrow_hbm, out_hbm,
                  edata_v, ew_v, rows_v, acc_sh,
                  rsem0, rsem1, isem0, isem1, isem2, isem3):
    cid = lax.axis_index("c")
    sid = lax.axis_index("s")
    rsems = (rsem0, rsem1)
    isems = (isem0, isem1, isem2, isem3)
    nch = jnp.where(cid == 0, NCH0, NCH1)
    cbase = jnp.where(cid == 0, sid * NCH0, CB1 + sid * NCH1)

    pltpu.sync_copy(zrow_hbm, acc_sh.at[pl.ds(sid * NPS, NPS)])

    @pl.when(sid == NS - 1)
    def _():
        pltpu.sync_copy(zrow_hbm.at[pl.ds(0, NTAIL)],
                        acc_sh.at[pl.ds(NS * NPS, NTAIL)])

    plsc.subcore_barrier()

    def fetch(c, ib):
        pltpu.async_copy(edata_hbm.at[cbase + c], edata_v.at[ib], isems[ib])
        pltpu.async_copy(ew_hbm.at[cbase + c], ew_v.at[ib], isems[ib])

    def wait_fetch(ib):
        pltpu.make_async_copy(edata_hbm.at[0], edata_v.at[ib],
                              isems[ib]).wait()
        pltpu.make_async_copy(ew_hbm.at[0], ew_v.at[ib],
                              isems[ib]).wait()

    def issue(ib, rb):
        pltpu.async_copy(y_hbm.at[edata_v.at[ib, 0]], rows_v.at[rb],
                         rsems[rb])

    def wait_rows(rb):
        pltpu.make_async_copy(y_hbm.at[pl.ds(0, K)], rows_v.at[rb],
                              rsems[rb]).wait()

    def scale(ib, rb):
        def scale_body(g, c2):
            ewv = ew_v[ib, pl.ds(g * 16, 16)]
            for l in range(16):
                w = lax.gather(ewv, jnp.full((16, 1), l, jnp.int32), _DNUMS,
                               slice_sizes=(1,),
                               mode=lax.GatherScatterMode.PROMISE_IN_BOUNDS)
                for j in range(D // 16):
                    rows_v[rb, g * 16 + l, pl.ds(j * 16, 16)] = \
                        rows_v[rb, g * 16 + l, pl.ds(j * 16, 16)] * w
            return c2

        lax.fori_loop(0, K // 16, scale_body, 0)

    def scatter(ib, rb):
        pltpu.sync_copy(rows_v.at[rb], acc_sh.at[edata_v.at[ib, 1]],
                        add=True)

    for ib in range(4):
        fetch(jnp.int32(ib), ib)
    for ib in range(2):
        wait_fetch(ib)
        issue(ib, ib)

    def quad_body(t, carry):
        c0 = 4 * t
        for u in range(4):
            c = c0 + u
            ib = u
            rb = u % 2
            wait_rows(rb)
            scale(ib, rb)
            scatter(ib, rb)

            @pl.when(c + 2 < nch)
            def _():
                wait_fetch((u + 2) % 4)
                issue((u + 2) % 4, rb)

            @pl.when(c + 4 < nch)
            def _():
                fetch(c + 4, ib)

        return carry

    lax.fori_loop(0, nch // 4, quad_body, 0)
    plsc.subcore_barrier()
    pltpu.sync_copy(acc_sh.at[pl.ds(sid * NPS, NPS)],
                    out_hbm.at[pl.ds(cid * N + sid * NPS, NPS)])

    @pl.when(sid == NS - 1)
    def _():
        pltpu.sync_copy(acc_sh.at[pl.ds(NS * NPS, NTAIL)],
                        out_hbm.at[pl.ds(cid * N + NS * NPS, NTAIL)])


def _aggregate(gidx, dst, ew, y):
    pad = E_PAD - E
    gidx_p = jnp.concatenate([gidx, jnp.zeros((pad,), jnp.int32)])
    dst_p = jnp.concatenate([dst.astype(jnp.int32), jnp.zeros((pad,), jnp.int32)])
    ew_p = jnp.concatenate([ew, jnp.zeros((pad,), jnp.float32)])
    edata = jnp.stack([gidx_p.reshape(TOTCH, K),
                       dst_p.reshape(TOTCH, K)], axis=1)
    zrow = jnp.zeros((NPS, D), jnp.float32)
    out = _sc_aggregate(edata, ew_p.reshape(TOTCH, K),
                        y.reshape(R * N, D), zrow)
    return out.reshape(2, N, D)


def kernel(unit_type, edge_index, edge_relation, edge_weight, node2graph,
           embedding,
           W_lin0, b_lin0, W_self0, b_self0,
           W_lin1, b_lin1, W_self1, b_self1,
           W_lin2, b_lin2, W_self2, b_self2):
    src = edge_index[0]
    dst = edge_index[1]
    gidx = edge_relation.astype(jnp.int32) * N + src.astype(jnp.int32)

    wl0 = W_lin0.reshape(R, D, D)
    wl1 = W_lin1.reshape(R, D, D)
    wl2 = W_lin2.reshape(R, D, D)
    b0 = (b_lin0 + b_self0).reshape(1, D)
    b1 = (b_lin1 + b_self1).reshape(1, D)
    b2 = (b_lin2 + b_self2).reshape(1, D)

    x0, y0 = _embed_call(unit_type.astype(jnp.int32), embedding, wl0)
    agg0 = _aggregate(gidx, dst, edge_weight, y0)
    x1, y1 = _combine_call(agg0, x0, W_self0, b0, wl1)
    agg1 = _aggregate(gidx, dst, edge_weight, y1)
    x2, y2 = _combine_call(agg1, x1, W_self1, b1, wl2)
    agg2 = _aggregate(gidx, dst, edge_weight, y2)
    node_feature, graph_feature = _final_call(agg2, x2, W_self2, b2,
                                              node2graph.astype(jnp.int32))
    return graph_feature, node_feature


# speedup vs baseline: 2.3525x; 1.0107x over previous
import functools

import jax
import jax.numpy as jnp
from jax import lax
from jax.experimental import pallas as pl
from jax.experimental.pallas import tpu as pltpu
from jax.experimental.pallas import tpu_sc as plsc

N = 10000
E = 320000
D = 128
R = 7
NU = 200
NG = 64

BN = 2000
NB = N // BN

NC = 2
NS = 16
NW = NC * NS
K = 128
NCH0 = 128
NCH1 = 32
TOTCH = NS * (NCH0 + NCH1)
E_PAD = TOTCH * K
CB1 = NS * NCH0
NPS = 624
NTAIL = N - NS * NPS


def _embed_body(ut_ref, emb_ref, wlin_ref, x_ref, y_ref):
    ut = ut_ref[0, 0, :]
    onehot = (ut[:, None] == lax.broadcasted_iota(jnp.int32, (BN, NU), 1))
    x = jnp.dot(onehot.astype(jnp.float32), emb_ref[...],
                preferred_element_type=jnp.float32)
    x_ref[...] = x
    for r in range(R):
        y_ref[r] = jnp.dot(x, wlin_ref[r], preferred_element_type=jnp.float32)


def _embed_call(unit_type, embedding, wlin):
    return pl.pallas_call(
        _embed_body,
        grid=(NB,),
        in_specs=[
            pl.BlockSpec((1, 1, BN), lambda i: (i, 0, 0)),
            pl.BlockSpec((NU, D), lambda i: (0, 0)),
            pl.BlockSpec((R, D, D), lambda i: (0, 0, 0)),
        ],
        out_specs=[
            pl.BlockSpec((BN, D), lambda i: (i, 0)),
            pl.BlockSpec((R, BN, D), lambda i: (0, i, 0)),
        ],
        out_shape=[
            jax.ShapeDtypeStruct((N, D), jnp.float32),
            jax.ShapeDtypeStruct((R, N, D), jnp.float32),
        ],
    )(unit_type.reshape(NB, 1, BN), embedding, wlin)


def _combine_body(agg_ref, x_ref, wself_ref, b_ref, wlin_ref, h_ref, y_ref):
    upd = agg_ref[0] + agg_ref[1]
    h = upd + jnp.dot(x_ref[...], wself_ref[...],
                      preferred_element_type=jnp.float32)
    h = jnp.maximum(h + b_ref[0], 0.0) + x_ref[...]
    h_ref[...] = h
    for r in range(R):
        y_ref[r] = jnp.dot(h, wlin_ref[r], preferred_element_type=jnp.float32)


def _combine_call(agg, x, wself, b, wlin_next):
    return pl.pallas_call(
        _combine_body,
        grid=(NB,),
        in_specs=[
            pl.BlockSpec((2, BN, D), lambda i: (0, i, 0)),
            pl.BlockSpec((BN, D), lambda i: (i, 0)),
            pl.BlockSpec((D, D), lambda i: (0, 0)),
            pl.BlockSpec((1, D), lambda i: (0, 0)),
            pl.BlockSpec((R, D, D), lambda i: (0, 0, 0)),
        ],
        out_specs=[
            pl.BlockSpec((BN, D), lambda i: (i, 0)),
            pl.BlockSpec((R, BN, D), lambda i: (0, i, 0)),
        ],
        out_shape=[
            jax.ShapeDtypeStruct((N, D), jnp.float32),
            jax.ShapeDtypeStruct((R, N, D), jnp.float32),
        ],
    )(agg, x, wself, b, wlin_next)


def _final_body(agg_ref, x_ref, wself_ref, b_ref, n2g_ref, h_ref, gf_ref):
    i = pl.program_id(0)
    upd = agg_ref[0] + agg_ref[1]
    h = upd + jnp.dot(x_ref[...], wself_ref[...],
                      preferred_element_type=jnp.float32)
    h = jnp.maximum(h + b_ref[0], 0.0) + x_ref[...]
    h_ref[...] = h
    n2g = n2g_ref[0, 0, :]
    onehot = (n2g[:, None] == lax.broadcasted_iota(jnp.int32, (BN, NG), 1))
    contrib = lax.dot_general(onehot.astype(jnp.float32), h,
                              (((0,), (0,)), ((), ())),
                              preferred_element_type=jnp.float32)

    @pl.when(i == 0)
    def _():
        gf_ref[...] = jnp.zeros_like(gf_ref)

    gf_ref[...] += contrib


def _final_call(agg, x, wself, b, node2graph):
    return pl.pallas_call(
        _final_body,
        grid=(NB,),
        in_specs=[
            pl.BlockSpec((2, BN, D), lambda i: (0, i, 0)),
            pl.BlockSpec((BN, D), lambda i: (i, 0)),
            pl.BlockSpec((D, D), lambda i: (0, 0)),
            pl.BlockSpec((1, D), lambda i: (0, 0)),
            pl.BlockSpec((1, 1, BN), lambda i: (i, 0, 0)),
        ],
        out_specs=[
            pl.BlockSpec((BN, D), lambda i: (i, 0)),
            pl.BlockSpec((NG, D), lambda i: (0, 0)),
        ],
        out_shape=[
            jax.ShapeDtypeStruct((N, D), jnp.float32),
            jax.ShapeDtypeStruct((NG, D), jnp.float32),
        ],
    )(agg, x, wself, b, node2graph.reshape(NB, 1, BN))


_sc_mesh = plsc.VectorSubcoreMesh(core_axis_name="c", subcore_axis_name="s")


_DNUMS = lax.GatherDimensionNumbers(
    offset_dims=(), collapsed_slice_dims=(0,), start_index_map=(0,))


@functools.partial(
    pl.kernel,
    mesh=_sc_mesh,
    out_type=jax.ShapeDtypeStruct((NC * N, D), jnp.float32),
    scratch_types=[
        pltpu.VMEM((4, 2, K), jnp.int32),
        pltpu.VMEM((4, K), jnp.float32),
        pltpu.VMEM((2, K, D), jnp.float32),
        pltpu.VMEM_SHARED((N, D), jnp.float32),
    ] + [pltpu.SemaphoreType.DMA] * 6,
)
def _sc_aggregate(edata_hbm, ew_hbm, y_hbm, zrow_hbm, out_hbm,
                  edata_v, ew_v, rows_v, acc_sh,
                  rsem0, rsem1, isem0, isem1, isem2, isem3):
    cid = lax.axis_index("c")
    sid = lax.axis_index("s")
    rsems = (rsem0, rsem1)
    isems = (isem0, isem1, isem2, isem3)
    nch = jnp.where(cid == 0, NCH0, NCH1)
    cbase = jnp.where(cid == 0, sid * NCH0, CB1 + sid * NCH1)

    pltpu.sync_copy(zrow_hbm, acc_sh.at[pl.ds(sid * NPS, NPS)])

    @pl.when(sid == NS - 1)
    def _():
        pltpu.sync_copy(zrow_hbm.at[pl.ds(0, NTAIL)],
                        acc_sh.at[pl.ds(NS * NPS, NTAIL)])

    plsc.subcore_barrier()

    def fetch(c, ib):
        pltpu.async_copy(edata_hbm.at[cbase + c], edata_v.at[ib], isems[ib])
        pltpu.async_copy(ew_hbm.at[cbase + c], ew_v.at[ib], isems[ib])

    def wait_fetch(ib):
        pltpu.make_async_copy(edata_hbm.at[0], edata_v.at[ib],
                              isems[ib]).wait()
        pltpu.make_async_copy(ew_hbm.at[0], ew_v.at[ib],
                              isems[ib]).wait()

    def issue(ib, rb):
        pltpu.async_copy(y_hbm.at[edata_v.at[ib, 0]], rows_v.at[rb],
                         rsems[rb])

    def wait_rows(rb):
        pltpu.make_async_copy(y_hbm.at[pl.ds(0, K)], rows_v.at[rb],
                              rsems[rb]).wait()

    def scale(ib, rb):
        def scale_body(g, c2):
            ewv = ew_v[ib, pl.ds(g * 16, 16)]
            for l in range(16):
                w = lax.gather(ewv, jnp.full((16, 1), l, jnp.int32), _DNUMS,
                               slice_sizes=(1,),
                               mode=lax.GatherScatterMode.PROMISE_IN_BOUNDS)
                for j in range(D // 16):
                    rows_v[rb, g * 16 + l, pl.ds(j * 16, 16)] = \
                        rows_v[rb, g * 16 + l, pl.ds(j * 16, 16)] * w
            return c2

        lax.fori_loop(0, K // 16, scale_body, 0)

    def scatter(ib, rb):
        pltpu.sync_copy(rows_v.at[rb], acc_sh.at[edata_v.at[ib, 1]],
                        add=True)

    for ib in range(4):
        fetch(jnp.int32(ib), ib)
    for ib in range(2):
        wait_fetch(ib)
        issue(ib, ib)

    def quad_body(t, carry):
        c0 = 4 * t
        for u in range(4):
            c = c0 + u
            ib = u
            rb = u % 2
            wait_rows(rb)
            scale(ib, rb)
            scatter(ib, rb)

            @pl.when(c + 2 < nch)
            def _():
                wait_fetch((u + 2) % 4)
                issue((u + 2) % 4, rb)

            @pl.when(c + 4 < nch)
            def _():
                fetch(c + 4, ib)

        return carry

    lax.fori_loop(0, nch // 4, quad_body, 0)
    plsc.subcore_barrier()
    pltpu.sync_copy(acc_sh.at[pl.ds(sid * NPS, NPS)],
                    out_hbm.at[pl.ds(cid * N + sid * NPS, NPS)])

    @pl.when(sid == NS - 1)
    def _():
        pltpu.sync_copy(acc_sh.at[pl.ds(NS * NPS, NTAIL)],
                        out_hbm.at[pl.ds(cid * N + NS * NPS, NTAIL)])


def _aggregate(gidx, dst, ew, y):
    pad = E_PAD - E
    gidx_p = jnp.concatenate([gidx, jnp.zeros((pad,), jnp.int32)])
    dst_p = jnp.concatenate([dst.astype(jnp.int32), jnp.zeros((pad,), jnp.int32)])
    ew_p = jnp.concatenate([ew, jnp.zeros((pad,), jnp.float32)])
    edata = jnp.stack([gidx_p.reshape(TOTCH, K),
                       dst_p.reshape(TOTCH, K)], axis=1)
    zrow = jnp.zeros((NPS, D), jnp.float32)
    out = _sc_aggregate(edata, ew_p.reshape(TOTCH, K),
                        y.reshape(R * N, D), zrow)
    return out.reshape(2, N, D)


def kernel(unit_type, edge_index, edge_relation, edge_weight, node2graph,
           embedding,
           W_lin0, b_lin0, W_self0, b_self0,
           W_lin1, b_lin1, W_self1, b_self1,
           W_lin2, b_lin2, W_self2, b_self2):
    src = edge_index[0]
    dst = edge_index[1]
    gidx = edge_relation.astype(jnp.int32) * N + src.astype(jnp.int32)

    wl0 = W_lin0.reshape(R, D, D)
    wl1 = W_lin1.reshape(R, D, D)
    wl2 = W_lin2.reshape(R, D, D)
    b0 = (b_lin0 + b_self0).reshape(1, D)
    b1 = (b_lin1 + b_self1).reshape(1, D)
    b2 = (b_lin2 + b_self2).reshape(1, D)

    x0, y0 = _embed_call(unit_type.astype(jnp.int32), embedding, wl0)
    agg0 = _aggregate(gidx, dst, edge_weight, y0)
    x1, y1 = _combine_call(agg0, x0, W_self0, b0, wl1)
    agg1 = _aggregate(gidx, dst, edge_weight, y1)
    x2, y2 = _combine_call(agg1, x1, W_self1, b1, wl2)
    agg2 = _aggregate(gidx, dst, edge_weight, y2)
    node_feature, graph_feature = _final_call(agg2, x2, W_self2, b2,
                                              node2graph.astype(jnp.int32))
    return graph_feature, node_feature


# speedup vs baseline: 2.5296x; 1.0753x over previous
import functools

import jax
import jax.numpy as jnp
from jax import lax
from jax.experimental import pallas as pl
from jax.experimental.pallas import tpu as pltpu
from jax.experimental.pallas import tpu_sc as plsc

N = 10000
E = 320000
D = 128
R = 7
NU = 200
NG = 64

BN = 2000
NB = N // BN

NC = 2
NS = 16
NW = NC * NS
K = 128
NCH0 = 144
NCH1 = 16
TOTCH = NS * (NCH0 + NCH1)
E_PAD = TOTCH * K
CB1 = NS * NCH0
NPS = 624
NTAIL = N - NS * NPS


def _embed_body(ut_ref, emb_ref, wlin_ref, x_ref, y_ref):
    ut = ut_ref[0, 0, :]
    onehot = (ut[:, None] == lax.broadcasted_iota(jnp.int32, (BN, NU), 1))
    x = jnp.dot(onehot.astype(jnp.float32), emb_ref[...],
                preferred_element_type=jnp.float32)
    x_ref[...] = x
    for r in range(R):
        y_ref[r] = jnp.dot(x, wlin_ref[r], preferred_element_type=jnp.float32)


def _embed_call(unit_type, embedding, wlin):
    return pl.pallas_call(
        _embed_body,
        grid=(NB,),
        in_specs=[
            pl.BlockSpec((1, 1, BN), lambda i: (i, 0, 0)),
            pl.BlockSpec((NU, D), lambda i: (0, 0)),
            pl.BlockSpec((R, D, D), lambda i: (0, 0, 0)),
        ],
        out_specs=[
            pl.BlockSpec((BN, D), lambda i: (i, 0)),
            pl.BlockSpec((R, BN, D), lambda i: (0, i, 0)),
        ],
        out_shape=[
            jax.ShapeDtypeStruct((N, D), jnp.float32),
            jax.ShapeDtypeStruct((R, N, D), jnp.float32),
        ],
    )(unit_type.reshape(NB, 1, BN), embedding, wlin)


def _combine_body(agg_ref, x_ref, wself_ref, b_ref, wlin_ref, h_ref, y_ref):
    upd = agg_ref[0] + agg_ref[1]
    h = upd + jnp.dot(x_ref[...], wself_ref[...],
                      preferred_element_type=jnp.float32)
    h = jnp.maximum(h + b_ref[0], 0.0) + x_ref[...]
    h_ref[...] = h
    for r in range(R):
        y_ref[r] = jnp.dot(h, wlin_ref[r], preferred_element_type=jnp.float32)


def _combine_call(agg, x, wself, b, wlin_next):
    return pl.pallas_call(
        _combine_body,
        grid=(NB,),
        in_specs=[
            pl.BlockSpec((2, BN, D), lambda i: (0, i, 0)),
            pl.BlockSpec((BN, D), lambda i: (i, 0)),
            pl.BlockSpec((D, D), lambda i: (0, 0)),
            pl.BlockSpec((1, D), lambda i: (0, 0)),
            pl.BlockSpec((R, D, D), lambda i: (0, 0, 0)),
        ],
        out_specs=[
            pl.BlockSpec((BN, D), lambda i: (i, 0)),
            pl.BlockSpec((R, BN, D), lambda i: (0, i, 0)),
        ],
        out_shape=[
            jax.ShapeDtypeStruct((N, D), jnp.float32),
            jax.ShapeDtypeStruct((R, N, D), jnp.float32),
        ],
    )(agg, x, wself, b, wlin_next)


def _final_body(agg_ref, x_ref, wself_ref, b_ref, n2g_ref, h_ref, gf_ref):
    i = pl.program_id(0)
    upd = agg_ref[0] + agg_ref[1]
    h = upd + jnp.dot(x_ref[...], wself_ref[...],
                      preferred_element_type=jnp.float32)
    h = jnp.maximum(h + b_ref[0], 0.0) + x_ref[...]
    h_ref[...] = h
    n2g = n2g_ref[0, 0, :]
    onehot = (n2g[:, None] == lax.broadcasted_iota(jnp.int32, (BN, NG), 1))
    contrib = lax.dot_general(onehot.astype(jnp.float32), h,
                              (((0,), (0,)), ((), ())),
                              preferred_element_type=jnp.float32)

    @pl.when(i == 0)
    def _():
        gf_ref[...] = jnp.zeros_like(gf_ref)

    gf_ref[...] += contrib


def _final_call(agg, x, wself, b, node2graph):
    return pl.pallas_call(
        _final_body,
        grid=(NB,),
        in_specs=[
            pl.BlockSpec((2, BN, D), lambda i: (0, i, 0)),
            pl.BlockSpec((BN, D), lambda i: (i, 0)),
            pl.BlockSpec((D, D), lambda i: (0, 0)),
            pl.BlockSpec((1, D), lambda i: (0, 0)),
            pl.BlockSpec((1, 1, BN), lambda i: (i, 0, 0)),
        ],
        out_specs=[
            pl.BlockSpec((BN, D), lambda i: (i, 0)),
            pl.BlockSpec((NG, D), lambda i: (0, 0)),
        ],
        out_shape=[
            jax.ShapeDtypeStruct((N, D), jnp.float32),
            jax.ShapeDtypeStruct((NG, D), jnp.float32),
        ],
    )(agg, x, wself, b, node2graph.reshape(NB, 1, BN))


_sc_mesh = plsc.VectorSubcoreMesh(core_axis_name="c", subcore_axis_name="s")


_DNUMS = lax.GatherDimensionNumbers(
    offset_dims=(), collapsed_slice_dims=(0,), start_index_map=(0,))


@functools.partial(
    pl.kernel,
    mesh=_sc_mesh,
    out_type=jax.ShapeDtypeStruct((NC * N, D), jnp.float32),
    scratch_types=[
        pltpu.VMEM((4, 2, K), jnp.int32),
        pltpu.VMEM((4, K), jnp.float32),
        pltpu.VMEM((2, K, D), jnp.float32),
        pltpu.VMEM_SHARED((N, D), jnp.float32),
    ] + [pltpu.SemaphoreType.DMA] * 6,
)
def _sc_aggregate(edata_hbm, ew_hbm, y_hbm, zrow_hbm, out_hbm,
                  edata_v, ew_v, rows_v, acc_sh,
                  rsem0, rsem1, isem0, isem1, isem2, isem3):
    cid = lax.axis_index("c")
    sid = lax.axis_index("s")
    rsems = (rsem0, rsem1)
    isems = (isem0, isem1, isem2, isem3)
    nch = jnp.where(cid == 0, NCH0, NCH1)
    cbase = jnp.where(cid == 0, sid * NCH0, CB1 + sid * NCH1)

    pltpu.sync_copy(zrow_hbm, acc_sh.at[pl.ds(sid * NPS, NPS)])

    @pl.when(sid == NS - 1)
    def _():
        pltpu.sync_copy(zrow_hbm.at[pl.ds(0, NTAIL)],
                        acc_sh.at[pl.ds(NS * NPS, NTAIL)])

    plsc.subcore_barrier()

    def fetch(c, ib):
        pltpu.async_copy(edata_hbm.at[cbase + c], edata_v.at[ib], isems[ib])
        pltpu.async_copy(ew_hbm.at[cbase + c], ew_v.at[ib], isems[ib])

    def wait_fetch(ib):
        pltpu.make_async_copy(edata_hbm.at[0], edata_v.at[ib],
                              isems[ib]).wait()
        pltpu.make_async_copy(ew_hbm.at[0], ew_v.at[ib],
                              isems[ib]).wait()

    def issue(ib, rb):
        pltpu.async_copy(y_hbm.at[edata_v.at[ib, 0]], rows_v.at[rb],
                         rsems[rb])

    def wait_rows(rb):
        pltpu.make_async_copy(y_hbm.at[pl.ds(0, K)], rows_v.at[rb],
                              rsems[rb]).wait()

    def scale(ib, rb):
        def scale_body(g, c2):
            ewv = ew_v[ib, pl.ds(g * 16, 16)]
            for l in range(16):
                w = lax.gather(ewv, jnp.full((16, 1), l, jnp.int32), _DNUMS,
                               slice_sizes=(1,),
                               mode=lax.GatherScatterMode.PROMISE_IN_BOUNDS)
                for j in range(D // 16):
                    rows_v[rb, g * 16 + l, pl.ds(j * 16, 16)] = \
                        rows_v[rb, g * 16 + l, pl.ds(j * 16, 16)] * w
            return c2

        lax.fori_loop(0, K // 16, scale_body, 0)

    def scatter(ib, rb):
        pltpu.sync_copy(rows_v.at[rb], acc_sh.at[edata_v.at[ib, 1]],
                        add=True)

    for ib in range(4):
        fetch(jnp.int32(ib), ib)
    for ib in range(2):
        wait_fetch(ib)
        issue(ib, ib)

    def quad_body(t, carry):
        c0 = 4 * t
        for u in range(4):
            c = c0 + u
            ib = u
            rb = u % 2
            wait_rows(rb)
            scale(ib, rb)
            scatter(ib, rb)

            @pl.when(c + 2 < nch)
            def _():
                wait_fetch((u + 2) % 4)
                issue((u + 2) % 4, rb)

            @pl.when(c + 4 < nch)
            def _():
                fetch(c + 4, ib)

        return carry

    lax.fori_loop(0, nch // 4, quad_body, 0)
    plsc.subcore_barrier()
    pltpu.sync_copy(acc_sh.at[pl.ds(sid * NPS, NPS)],
                    out_hbm.at[pl.ds(cid * N + sid * NPS, NPS)])

    @pl.when(sid == NS - 1)
    def _():
        pltpu.sync_copy(acc_sh.at[pl.ds(NS * NPS, NTAIL)],
                        out_hbm.at[pl.ds(cid * N + NS * NPS, NTAIL)])


def _aggregate(gidx, dst, ew, y):
    pad = E_PAD - E
    gidx_p = jnp.concatenate([gidx, jnp.zeros((pad,), jnp.int32)])
    dst_p = jnp.concatenate([dst.astype(jnp.int32), jnp.zeros((pad,), jnp.int32)])
    ew_p = jnp.concatenate([ew, jnp.zeros((pad,), jnp.float32)])
    edata = jnp.stack([gidx_p.reshape(TOTCH, K),
                       dst_p.reshape(TOTCH, K)], axis=1)
    zrow = jnp.zeros((NPS, D), jnp.float32)
    out = _sc_aggregate(edata, ew_p.reshape(TOTCH, K),
                        y.reshape(R * N, D), zrow)
    return out.reshape(2, N, D)


def kernel(unit_type, edge_index, edge_relation, edge_weight, node2graph,
           embedding,
           W_lin0, b_lin0, W_self0, b_self0,
           W_lin1, b_lin1, W_self1, b_self1,
           W_lin2, b_lin2, W_self2, b_self2):
    src = edge_index[0]
    dst = edge_index[1]
    gidx = edge_relation.astype(jnp.int32) * N + src.astype(jnp.int32)

    wl0 = W_lin0.reshape(R, D, D)
    wl1 = W_lin1.reshape(R, D, D)
    wl2 = W_lin2.reshape(R, D, D)
    b0 = (b_lin0 + b_self0).reshape(1, D)
    b1 = (b_lin1 + b_self1).reshape(1, D)
    b2 = (b_lin2 + b_self2).reshape(1, D)

    x0, y0 = _embed_call(unit_type.astype(jnp.int32), embedding, wl0)
    agg0 = _aggregate(gidx, dst, edge_weight, y0)
    x1, y1 = _combine_call(agg0, x0, W_self0, b0, wl1)
    agg1 = _aggregate(gidx, dst, edge_weight, y1)
    x2, y2 = _combine_call(agg1, x1, W_self1, b1, wl2)
    agg2 = _aggregate(gidx, dst, edge_weight, y2)
    node_feature, graph_feature = _final_call(agg2, x2, W_self2, b2,
                                              node2graph.astype(jnp.int32))
    return graph_feature, node_feature
